# Initial kernel scaffold; baseline (speedup 1.0000x reference)
import functools

import jax
import jax.numpy as jnp
from jax import lax
from jax.experimental import pallas as pl
from jax.experimental.pallas import tpu as pltpu
from jax.experimental.pallas import tpu_sc as plsc

_IW = 128
_K = 8


@functools.cache
def _make_gather(n: int, v: int, d: int):
  info = plsc.get_sparse_core_info()
  nc, ns = info.num_cores, info.num_subcores
  nw = nc * ns
  assert n % (nw * _IW) == 0, (n, nw)
  rows_per_w = n // (nw * _IW)
  assert rows_per_w % _K == 0, rows_per_w
  groups = rows_per_w // _K
  chunk = _K * _IW

  mesh = plsc.VectorSubcoreMesh(core_axis_name="c", subcore_axis_name="s")

  @functools.partial(
      pl.kernel,
      mesh=mesh,
      out_type=jax.ShapeDtypeStruct((n, d), jnp.float32),
      scratch_types=[
          pltpu.VMEM((rows_per_w, _IW), jnp.int32),
          pltpu.VMEM((chunk, d), jnp.float32),
          pltpu.SemaphoreType.DMA,
      ],
  )
  def gather(idx_hbm, table_hbm, out_hbm, idx_v, rows_v, sem):
    wid = lax.axis_index("s") * nc + lax.axis_index("c")
    row0 = wid * rows_per_w
    pltpu.sync_copy(idx_hbm.at[pl.ds(row0, rows_per_w), :], idx_v)

    def body(g, carry):
      copies = [
          pltpu.async_copy(
              table_hbm.at[idx_v.at[g * _K + j]],
              rows_v.at[pl.ds(j * _IW, _IW), :],
              sem,
          )
          for j in range(_K)
      ]
      for c in copies:
        c.wait()
      out0 = (row0 + g * _K) * _IW
      pltpu.sync_copy(rows_v, out_hbm.at[pl.ds(out0, chunk), :])
      return carry

    lax.fori_loop(0, groups, body, 0)

  return gather


def kernel(ids, table):
  b, f = ids.shape
  v, d = table.shape
  n = b * f
  idx2d = ids.reshape(n // _IW, _IW)
  out = _make_gather(n, v, d)(idx2d, table)
  return out.reshape(b, f, d)


# speedup vs baseline: 5.6575x; 5.6575x over previous
import functools

import jax
import jax.numpy as jnp
from jax import lax
from jax.experimental import pallas as pl
from jax.experimental.pallas import tpu as pltpu
from jax.experimental.pallas import tpu_sc as plsc

_IW = 128
_K = 8


@functools.cache
def _make_gather(n: int, v: int, d: int):
  info = plsc.get_sparse_core_info()
  nc, ns = info.num_cores, info.num_subcores
  nw = nc * ns
  assert n % (nw * _IW) == 0, (n, nw)
  rows_per_w = n // (nw * _IW)
  assert rows_per_w % _K == 0, rows_per_w
  groups = rows_per_w // _K
  chunk = _K * _IW

  mesh = plsc.VectorSubcoreMesh(core_axis_name="c", subcore_axis_name="s")

  @functools.partial(
      pl.kernel,
      mesh=mesh,
      out_type=jax.ShapeDtypeStruct((n, d), jnp.float32),
      compiler_params=pltpu.CompilerParams(use_tc_tiling_on_sc=False),
      scratch_types=[
          pltpu.VMEM((rows_per_w, _IW), jnp.int32),
          pltpu.VMEM((chunk, d), jnp.float32),
          pltpu.SemaphoreType.DMA,
      ],
  )
  def gather(idx_hbm, table_hbm, out_hbm, idx_v, rows_v, sem):
    wid = lax.axis_index("s") * nc + lax.axis_index("c")
    row0 = wid * rows_per_w
    pltpu.sync_copy(idx_hbm.at[pl.ds(row0, rows_per_w), :], idx_v)

    def body(g, carry):
      copies = [
          pltpu.async_copy(
              table_hbm.at[idx_v.at[g * _K + j]],
              rows_v.at[pl.ds(j * _IW, _IW), :],
              sem,
          )
          for j in range(_K)
      ]
      for c in copies:
        c.wait()
      out0 = (row0 + g * _K) * _IW
      pltpu.sync_copy(rows_v, out_hbm.at[pl.ds(out0, chunk), :])
      return carry

    lax.fori_loop(0, groups, body, 0)

  return gather


def kernel(ids, table):
  b, f = ids.shape
  v, d = table.shape
  n = b * f
  idx2d = ids.reshape(n // _IW, _IW)
  out = _make_gather(n, v, d)(idx2d, table)
  return out.reshape(b, f, d)


# speedup vs baseline: 5.7181x; 1.0107x over previous
import functools

import jax
import jax.numpy as jnp
from jax import lax
from jax.experimental import pallas as pl
from jax.experimental.pallas import tpu as pltpu
from jax.experimental.pallas import tpu_sc as plsc

_IW = 128
_K = 13
_G = 8


@functools.cache
def _make_gather(n: int, v: int, d: int):
  info = plsc.get_sparse_core_info()
  nc, ns = info.num_cores, info.num_subcores
  nw = nc * ns
  rows_per_w = n // (nw * _IW)
  assert rows_per_w == _K * _G, rows_per_w
  chunk = _K * _IW

  mesh = plsc.VectorSubcoreMesh(core_axis_name="c", subcore_axis_name="s")

  @functools.partial(
      pl.kernel,
      mesh=mesh,
      out_type=jax.ShapeDtypeStruct((n, d), jnp.float32),
      compiler_params=pltpu.CompilerParams(use_tc_tiling_on_sc=False),
      scratch_types=[
          pltpu.VMEM((rows_per_w, _IW), jnp.int32),
          pltpu.VMEM((chunk, d), jnp.float32),
          pltpu.VMEM((chunk, d), jnp.float32),
          pltpu.SemaphoreType.DMA,
          pltpu.SemaphoreType.DMA,
          pltpu.SemaphoreType.DMA,
          pltpu.SemaphoreType.DMA,
      ],
  )
  def gather(idx_hbm, table_hbm, out_hbm, idx_v, buf0, buf1, sg0, sg1,
             sw0, sw1):
    wid = lax.axis_index("s") * nc + lax.axis_index("c")
    row0 = wid * rows_per_w
    out_base = row0 * _IW
    pltpu.sync_copy(idx_hbm.at[pl.ds(row0, rows_per_w), :], idx_v)

    def fire_gathers(g, buf, sem):
      for j in range(_K):
        pltpu.async_copy(
            table_hbm.at[idx_v.at[g * _K + j]],
            buf.at[pl.ds(j * _IW, _IW), :],
            sem,
        )

    def wait_gathers(buf, sem):
      for j in range(_K):
        pltpu.make_async_copy(
            table_hbm.at[idx_v.at[j]],
            buf.at[pl.ds(j * _IW, _IW), :],
            sem,
        ).wait()

    def fire_wb(g, buf, sem):
      pltpu.async_copy(buf, out_hbm.at[pl.ds(out_base + g * chunk, chunk), :],
                       sem)

    def wait_wb(buf, sem):
      pltpu.make_async_copy(buf, out_hbm.at[pl.ds(out_base, chunk), :],
                            sem).wait()

    fire_gathers(0, buf0, sg0)
    fire_gathers(1, buf1, sg1)
    wait_gathers(buf0, sg0)
    fire_wb(0, buf0, sw0)

    def body(j, carry):
      g = 2 * j + 2
      wait_wb(buf0, sw0)
      fire_gathers(g, buf0, sg0)
      wait_gathers(buf1, sg1)
      fire_wb(g - 1, buf1, sw1)
      wait_wb(buf1, sw1)
      fire_gathers(g + 1, buf1, sg1)
      wait_gathers(buf0, sg0)
      fire_wb(g, buf0, sw0)
      return carry

    lax.fori_loop(0, _G // 2 - 1, body, 0)

    wait_gathers(buf1, sg1)
    fire_wb(_G - 1, buf1, sw1)
    wait_wb(buf0, sw0)
    wait_wb(buf1, sw1)

  return gather


def kernel(ids, table):
  b, f = ids.shape
  v, d = table.shape
  n = b * f
  idx2d = ids.reshape(n // _IW, _IW)
  out = _make_gather(n, v, d)(idx2d, table)
  return out.reshape(b, f, d)


# speedup vs baseline: 7.8827x; 1.3785x over previous
import functools

import jax
import jax.numpy as jnp
from jax import lax
from jax.experimental import pallas as pl
from jax.experimental.pallas import tpu as pltpu
from jax.experimental.pallas import tpu_sc as plsc

_IW = 128
_RPU = 4
_CB = _RPU * _IW
_TC = 1024


def _worker_id(nc):
  return lax.axis_index("s") * nc + lax.axis_index("c")


@functools.cache
def _make_pack(v: int, d: int):
  info = plsc.get_sparse_core_info()
  nc, ns, nl = info.num_cores, info.num_subcores, info.num_lanes
  nw = nc * ns
  nch = v // _TC
  tail = v - nch * _TC
  iters = (nch + nw - 1) // nw
  rpc = _TC * d // 128

  mesh = plsc.VectorSubcoreMesh(core_axis_name="c", subcore_axis_name="s")

  @functools.partial(
      pl.kernel,
      mesh=mesh,
      out_type=jax.ShapeDtypeStruct((v * d // 128, 128), jnp.float32),
      compiler_params=pltpu.CompilerParams(needs_layout_passes=False),
      scratch_types=[
          pltpu.VMEM((d, _TC), jnp.float32),
          pltpu.VMEM((rpc, 128), jnp.float32),
          pltpu.SemaphoreType.DMA,
      ],
  )
  def pack(tt_hbm, tail_hbm, out_hbm, buf, outv, swb):
    wid = _worker_id(nc)
    lane = lax.iota(jnp.int32, nl)
    ratio = 128 // d

    def do_chunk(row0, nrows, packed0):
      pltpu.sync_copy(tt_hbm.at[:, pl.ds(row0, nrows)],
                      buf.at[:, pl.ds(0, nrows)])

      def tj(j, cc):
        rows = j * nl + lane
        for dd in range(d):
          cols = (lane + dd) & (d - 1)
          vals = plsc.load_gather(buf, [cols, rows])
          plsc.store_scatter(
              outv, [rows // ratio, (rows % ratio) * d + cols], vals)
        return cc

      lax.fori_loop(0, nrows // nl, tj, 0)
      pltpu.async_copy(outv.at[pl.ds(0, nrows * d // 128), :],
                       out_hbm.at[pl.ds(packed0, nrows * d // 128), :], swb)

    def wait_wb(nrows):
      pltpu.make_async_copy(
          outv.at[pl.ds(0, nrows * d // 128), :],
          out_hbm.at[pl.ds(0, nrows * d // 128), :], swb).wait()

    def body(i, carry):
      c = wid + i * nw

      @pl.when(c < nch)
      def _():
        @pl.when(i > 0)
        def _():
          wait_wb(_TC)

        do_chunk(c * _TC, _TC, c * rpc)

      return carry

    lax.fori_loop(0, iters, body, 0)

    wait_wb(_TC)

    if tail:
      trows = tail * d // 128

      @pl.when(wid == 0)
      def _():
        pltpu.sync_copy(tail_hbm, outv.at[pl.ds(0, trows), :])
        pltpu.sync_copy(outv.at[pl.ds(0, trows), :],
                        out_hbm.at[pl.ds(nch * rpc, trows), :])

  return pack


@functools.cache
def _make_gather(b: int, f: int, v: int, d: int):
  info = plsc.get_sparse_core_info()
  nc, ns, nl = info.num_cores, info.num_subcores, info.num_lanes
  nw = nc * ns
  assert b == nw * _CB and d % nl == 0 and nl == 16

  mesh = plsc.VectorSubcoreMesh(core_axis_name="c", subcore_axis_name="s")

  @functools.partial(
      pl.kernel,
      mesh=mesh,
      out_type=jax.ShapeDtypeStruct((f, d, b), jnp.float32),
      compiler_params=pltpu.CompilerParams(
          use_tc_tiling_on_sc=False, needs_layout_passes=False),
      scratch_types=[
          pltpu.VMEM((f, _RPU, _IW), jnp.int32),
          pltpu.VMEM((_CB, d), jnp.float32),
          pltpu.VMEM((_CB, d), jnp.float32),
          pltpu.VMEM((d, _CB), jnp.float32),
          pltpu.SemaphoreType.DMA,
          pltpu.SemaphoreType.DMA,
      ],
  )
  def gather(idx_hbm, table_hbm, out_hbm, idx_all, buf0, buf1, out_v,
             sg0, sg1):
    wid = _worker_id(nc)
    b0 = wid * _CB
    pltpu.sync_copy(idx_hbm.at[:, pl.ds(wid * _RPU, _RPU), :], idx_all)

    bufs = (buf0, buf1)
    sems = (sg0, sg1)

    def fire(u, p):
      for r in range(_RPU):
        pltpu.async_copy(
            table_hbm.at[idx_all.at[u, r]],
            bufs[p].at[pl.ds(r * _IW, _IW), :],
            sems[p],
        )

    def wait(p):
      for r in range(_RPU):
        pltpu.make_async_copy(
            table_hbm.at[idx_all.at[0, r]],
            bufs[p].at[pl.ds(r * _IW, _IW), :],
            sems[p],
        ).wait()

    lane = lax.iota(jnp.int32, nl)

    def transpose(p):
      buf = bufs[p]

      def tj(j, carry):
        rows = j * nl + lane
        for dd in range(d):
          cols = (lane + dd) & (d - 1)
          vals = plsc.load_gather(buf, [rows, cols])
          plsc.store_scatter(out_v, [cols, rows], vals)
        return carry

      lax.fori_loop(0, _CB // nl, tj, 0)

    def unit(u, p, prefetch):
      wait(p)

      @pl.when(prefetch)
      def _():
        fire(u + 1, 1 - p)

      transpose(p)
      pltpu.sync_copy(out_v, out_hbm.at[u, :, pl.ds(b0, _CB)])

    npairs = f // 2
    fire(0, 0)

    def body(j, carry):
      unit(2 * j, 0, jnp.bool_(True))
      unit(2 * j + 1, 1, j < npairs - 1)
      return carry

    lax.fori_loop(0, npairs, body, 0)

  return gather


def kernel(ids, table):
  b, f = ids.shape
  v, d = table.shape
  ids3 = ids.T.reshape(f, b // _IW, _IW)
  nfull = (v // _TC) * _TC
  tail_rm = table[nfull:, :].reshape((v - nfull) * d // 128, 128)
  packed = _make_pack(v, d)(table.T, tail_rm)
  table_rm = packed.reshape(v, d)
  out_t = _make_gather(b, f, v, d)(ids3, table_rm)
  return jnp.transpose(out_t, (2, 0, 1))


# speedup vs baseline: 10.3740x; 1.3160x over previous
import functools

import jax
import jax.numpy as jnp
from jax import lax
from jax.experimental import pallas as pl
from jax.experimental.pallas import tpu as pltpu
from jax.experimental.pallas import tpu_sc as plsc

_IW = 128
_RPU = 4
_CB = _RPU * _IW
_TC = 768


def _worker_id(nc):
  return lax.axis_index("s") * nc + lax.axis_index("c")


@functools.cache
def _make_pack(v: int, d: int):
  info = plsc.get_sparse_core_info()
  nc, ns, nl = info.num_cores, info.num_subcores, info.num_lanes
  nw = nc * ns
  nch = v // _TC
  tail = v - nch * _TC
  pairs = (nch + 2 * nw - 1) // (2 * nw)
  rpc = _TC * d // 128

  mesh = plsc.VectorSubcoreMesh(core_axis_name="c", subcore_axis_name="s")

  @functools.partial(
      pl.kernel,
      mesh=mesh,
      out_type=jax.ShapeDtypeStruct((v * d // 128, 128), jnp.float32),
      compiler_params=pltpu.CompilerParams(needs_layout_passes=False),
      scratch_types=[
          pltpu.VMEM((d, _TC), jnp.float32),
          pltpu.VMEM((d, _TC), jnp.float32),
          pltpu.VMEM((rpc, 128), jnp.float32),
          pltpu.VMEM((rpc, 128), jnp.float32),
          pltpu.SemaphoreType.DMA,
          pltpu.SemaphoreType.DMA,
          pltpu.SemaphoreType.DMA,
          pltpu.SemaphoreType.DMA,
      ],
  )
  def pack(tt_hbm, tail_hbm, out_hbm, buf0, buf1, ov0, ov1,
           ss0, ss1, sw0, sw1):
    wid = _worker_id(nc)
    lane = lax.iota(jnp.int32, nl)
    cols = [(lane + dd) & (d - 1) for dd in range(d)]
    bufs, ovs = (buf0, buf1), (ov0, ov1)
    ssems, wsems = (ss0, ss1), (sw0, sw1)

    def fire_stage(c, p):
      pltpu.async_copy(tt_hbm.at[:, pl.ds(c * _TC, _TC)], bufs[p], ssems[p])

    def wait_stage(p):
      pltpu.make_async_copy(
          tt_hbm.at[:, pl.ds(0, _TC)], bufs[p], ssems[p]).wait()

    def fire_wb(c, p):
      pltpu.async_copy(ovs[p], out_hbm.at[pl.ds(c * rpc, rpc), :], wsems[p])

    def wait_wb(p):
      pltpu.make_async_copy(
          ovs[p], out_hbm.at[pl.ds(0, rpc), :], wsems[p]).wait()

    def transpose(p):
      buf, outv = bufs[p], ovs[p]

      def tj(j, cc):
        rows = j * nl + lane
        rdiv = rows >> 2
        rmod = (rows & 3) << 5
        for dd in range(d):
          vals = plsc.load_gather(buf, [cols[dd], rows])
          plsc.store_scatter(outv, [rdiv, rmod + cols[dd]], vals)
        return cc

      lax.fori_loop(0, _TC // nl, tj, 0)

    def half(c_this, c_next2, p, first):
      @pl.when(c_this < nch)
      def _():
        wait_stage(p)

        @pl.when(jnp.logical_not(first))
        def _():
          wait_wb(p)

        transpose(p)
        fire_wb(c_this, p)

        @pl.when(c_next2 < nch)
        def _():
          fire_stage(c_next2, p)

    fire_stage(wid, 0)

    @pl.when(wid + nw < nch)
    def _():
      fire_stage(wid + nw, 1)

    def body(i, carry):
      ca = wid + (2 * i) * nw
      cb = wid + (2 * i + 1) * nw
      half(ca, wid + (2 * i + 2) * nw, 0, i == 0)
      half(cb, wid + (2 * i + 3) * nw, 1, i == 0)
      return carry

    lax.fori_loop(0, pairs, body, 0)
    wait_wb(0)
    wait_wb(1)

    if tail:
      trows = tail * d // 128

      @pl.when(wid == 0)
      def _():
        pltpu.sync_copy(tail_hbm, ov0.at[pl.ds(0, trows), :])
        pltpu.sync_copy(ov0.at[pl.ds(0, trows), :],
                        out_hbm.at[pl.ds(nch * rpc, trows), :])

  return pack


@functools.cache
def _make_gather(b: int, f: int, v: int, d: int):
  info = plsc.get_sparse_core_info()
  nc, ns, nl = info.num_cores, info.num_subcores, info.num_lanes
  nw = nc * ns
  assert b == nw * _CB and d % nl == 0 and nl == 16

  mesh = plsc.VectorSubcoreMesh(core_axis_name="c", subcore_axis_name="s")

  @functools.partial(
      pl.kernel,
      mesh=mesh,
      out_type=jax.ShapeDtypeStruct((f, d, b), jnp.float32),
      compiler_params=pltpu.CompilerParams(
          use_tc_tiling_on_sc=False, needs_layout_passes=False),
      scratch_types=[
          pltpu.VMEM((f, _RPU, _IW), jnp.int32),
          pltpu.VMEM((_CB, d), jnp.float32),
          pltpu.VMEM((_CB, d), jnp.float32),
          pltpu.VMEM((d, _CB), jnp.float32),
          pltpu.VMEM((d, _CB), jnp.float32),
          pltpu.SemaphoreType.DMA,
          pltpu.SemaphoreType.DMA,
          pltpu.SemaphoreType.DMA,
          pltpu.SemaphoreType.DMA,
      ],
  )
  def gather(idx_hbm, table_hbm, out_hbm, idx_all, buf0, buf1, ov0, ov1,
             sg0, sg1, sw0, sw1):
    wid = _worker_id(nc)
    b0 = wid * _CB
    pltpu.sync_copy(idx_hbm.at[:, pl.ds(wid * _RPU, _RPU), :], idx_all)

    bufs, ovs = (buf0, buf1), (ov0, ov1)
    gsems, wsems = (sg0, sg1), (sw0, sw1)
    lane = lax.iota(jnp.int32, nl)
    cols = [(lane + dd) & (d - 1) for dd in range(d)]

    def fire(u, p):
      for r in range(_RPU):
        pltpu.async_copy(
            table_hbm.at[idx_all.at[u, r]],
            bufs[p].at[pl.ds(r * _IW, _IW), :],
            gsems[p],
        )

    def wait(p):
      for r in range(_RPU):
        pltpu.make_async_copy(
            table_hbm.at[idx_all.at[0, r]],
            bufs[p].at[pl.ds(r * _IW, _IW), :],
            gsems[p],
        ).wait()

    def fire_wb(u, p):
      pltpu.async_copy(ovs[p], out_hbm.at[u, :, pl.ds(b0, _CB)], wsems[p])

    def wait_wb(p):
      pltpu.make_async_copy(
          ovs[p], out_hbm.at[0, :, pl.ds(b0, _CB)], wsems[p]).wait()

    def transpose(p):
      buf, outv = bufs[p], ovs[p]

      def tj(j, carry):
        rows = j * nl + lane
        for dd in range(d):
          vals = plsc.load_gather(buf, [rows, cols[dd]])
          plsc.store_scatter(outv, [cols[dd], rows], vals)
        return carry

      lax.fori_loop(0, _CB // nl, tj, 0)

    def unit(u, p, prefetch, first):
      wait(p)

      @pl.when(prefetch)
      def _():
        fire(u + 1, 1 - p)

      @pl.when(jnp.logical_not(first))
      def _():
        wait_wb(p)

      transpose(p)
      fire_wb(u, p)

    npairs = f // 2
    fire(0, 0)

    def body(j, carry):
      unit(2 * j, 0, jnp.bool_(True), j == 0)
      unit(2 * j + 1, 1, j < npairs - 1, j == 0)
      return carry

    lax.fori_loop(0, npairs, body, 0)
    wait_wb(0)
    wait_wb(1)

  return gather


def kernel(ids, table):
  b, f = ids.shape
  v, d = table.shape
  ids3 = ids.T.reshape(f, b // _IW, _IW)
  nfull = (v // _TC) * _TC
  tail_rm = table[nfull:, :].reshape((v - nfull) * d // 128, 128)
  packed = _make_pack(v, d)(table.T, tail_rm)
  table_rm = packed.reshape(v, d)
  out_t = _make_gather(b, f, v, d)(ids3, table_rm)
  return jnp.transpose(out_t, (2, 0, 1))


# speedup vs baseline: 11.9317x; 1.1502x over previous
import functools

import jax
import jax.numpy as jnp
from jax import lax
from jax.experimental import pallas as pl
from jax.experimental.pallas import tpu as pltpu
from jax.experimental.pallas import tpu_sc as plsc

_IW = 128
_RPU = 4
_CB = _RPU * _IW
_TC = 768


def _worker_id(nc):
  return lax.axis_index("s") * nc + lax.axis_index("c")


@functools.cache
def _make_pack(v: int, d: int):
  info = plsc.get_sparse_core_info()
  nc, ns, nl = info.num_cores, info.num_subcores, info.num_lanes
  nw = nc * ns
  nch = v // _TC
  tail = v - nch * _TC
  pairs = (nch + 2 * nw - 1) // (2 * nw)
  rpc = _TC * d // 128

  mesh = plsc.VectorSubcoreMesh(core_axis_name="c", subcore_axis_name="s")

  @functools.partial(
      pl.kernel,
      mesh=mesh,
      out_type=jax.ShapeDtypeStruct((v * d // 128, 128), jnp.float32),
      compiler_params=pltpu.CompilerParams(needs_layout_passes=False),
      scratch_types=[
          pltpu.VMEM((d, _TC), jnp.float32),
          pltpu.VMEM((d, _TC), jnp.float32),
          pltpu.VMEM((rpc, 128), jnp.float32),
          pltpu.VMEM((rpc, 128), jnp.float32),
          pltpu.SemaphoreType.DMA,
          pltpu.SemaphoreType.DMA,
          pltpu.SemaphoreType.DMA,
          pltpu.SemaphoreType.DMA,
      ],
  )
  def pack(tt_hbm, tail_hbm, out_hbm, buf0, buf1, ov0, ov1,
           ss0, ss1, sw0, sw1):
    wid = _worker_id(nc)
    lane = lax.iota(jnp.int32, nl)
    cols = [(lane + dd) & (d - 1) for dd in range(d)]
    bufs, ovs = (buf0, buf1), (ov0, ov1)
    ssems, wsems = (ss0, ss1), (sw0, sw1)

    def fire_stage(c, p):
      pltpu.async_copy(tt_hbm.at[:, pl.ds(c * _TC, _TC)], bufs[p], ssems[p])

    def wait_stage(p):
      pltpu.make_async_copy(
          tt_hbm.at[:, pl.ds(0, _TC)], bufs[p], ssems[p]).wait()

    def fire_wb(c, p):
      pltpu.async_copy(ovs[p], out_hbm.at[pl.ds(c * rpc, rpc), :], wsems[p])

    def wait_wb(p):
      pltpu.make_async_copy(
          ovs[p], out_hbm.at[pl.ds(0, rpc), :], wsems[p]).wait()

    def transpose(p):
      buf, outv = bufs[p], ovs[p]

      def tj(j, cc):
        rows = j * nl + lane
        rdiv = rows >> 2
        rmod = (rows & 3) << 5
        for dd in range(d):
          vals = plsc.load_gather(buf, [cols[dd], rows])
          plsc.store_scatter(outv, [rdiv, rmod + cols[dd]], vals)
        return cc

      lax.fori_loop(0, _TC // nl, tj, 0)

    def half(c_this, c_next2, p, first):
      @pl.when(c_this < nch)
      def _():
        wait_stage(p)

        @pl.when(jnp.logical_not(first))
        def _():
          wait_wb(p)

        transpose(p)
        fire_wb(c_this, p)

        @pl.when(c_next2 < nch)
        def _():
          fire_stage(c_next2, p)

    fire_stage(wid, 0)

    @pl.when(wid + nw < nch)
    def _():
      fire_stage(wid + nw, 1)

    def body(i, carry):
      ca = wid + (2 * i) * nw
      cb = wid + (2 * i + 1) * nw
      half(ca, wid + (2 * i + 2) * nw, 0, i == 0)
      half(cb, wid + (2 * i + 3) * nw, 1, i == 0)
      return carry

    lax.fori_loop(0, pairs, body, 0)
    wait_wb(0)
    wait_wb(1)

    if tail:
      trows = tail * d // 128

      @pl.when(wid == 0)
      def _():
        pltpu.sync_copy(tail_hbm, ov0.at[pl.ds(0, trows), :])
        pltpu.sync_copy(ov0.at[pl.ds(0, trows), :],
                        out_hbm.at[pl.ds(nch * rpc, trows), :])

  return pack


@functools.cache
def _make_gather(b: int, f: int, v: int, d: int):
  info = plsc.get_sparse_core_info()
  nc, ns, nl = info.num_cores, info.num_subcores, info.num_lanes
  nw = nc * ns
  assert b == nw * _CB and d % nl == 0 and nl == 16

  mesh = plsc.VectorSubcoreMesh(core_axis_name="c", subcore_axis_name="s")

  @functools.partial(
      pl.kernel,
      mesh=mesh,
      out_type=jax.ShapeDtypeStruct((f, d // 8, b // 128, 8, 128),
                                    jnp.float32),
      compiler_params=pltpu.CompilerParams(
          use_tc_tiling_on_sc=False, needs_layout_passes=False),
      scratch_types=[
          pltpu.VMEM((f, _RPU, _IW), jnp.int32),
          pltpu.VMEM((_CB, d), jnp.float32),
          pltpu.VMEM((_CB, d), jnp.float32),
          pltpu.VMEM((d // 8, _CB // 128, 8, 128), jnp.float32),
          pltpu.VMEM((d // 8, _CB // 128, 8, 128), jnp.float32),
          pltpu.SemaphoreType.DMA,
          pltpu.SemaphoreType.DMA,
          pltpu.SemaphoreType.DMA,
          pltpu.SemaphoreType.DMA,
      ],
  )
  def gather(idx_hbm, table_hbm, out_hbm, idx_all, buf0, buf1, ov0, ov1,
             sg0, sg1, sw0, sw1):
    wid = _worker_id(nc)
    b0 = wid * _CB
    pltpu.sync_copy(idx_hbm.at[:, pl.ds(wid * _RPU, _RPU), :], idx_all)

    bufs, ovs = (buf0, buf1), (ov0, ov1)
    gsems, wsems = (sg0, sg1), (sw0, sw1)
    lane = lax.iota(jnp.int32, nl)
    cols = [(lane + dd) & (d - 1) for dd in range(d)]

    def fire(u, p):
      for r in range(_RPU):
        pltpu.async_copy(
            table_hbm.at[idx_all.at[u, r]],
            bufs[p].at[pl.ds(r * _IW, _IW), :],
            gsems[p],
        )

    def wait(p):
      for r in range(_RPU):
        pltpu.make_async_copy(
            table_hbm.at[idx_all.at[0, r]],
            bufs[p].at[pl.ds(r * _IW, _IW), :],
            gsems[p],
        ).wait()

    tb0 = wid * (_CB // 128)
    tdv = [c >> 3 for c in cols]
    d8v = [c & 7 for c in cols]

    def fire_wb(u, p):
      pltpu.async_copy(
          ovs[p], out_hbm.at[u, :, pl.ds(tb0, _CB // 128), :, :], wsems[p])

    def wait_wb(p):
      pltpu.make_async_copy(
          ovs[p], out_hbm.at[0, :, pl.ds(0, _CB // 128), :, :],
          wsems[p]).wait()

    def transpose(p):
      buf, outv = bufs[p], ovs[p]

      def tj(j, carry):
        rows = j * nl + lane
        tb = rows >> 7
        b128 = rows & 127
        for dd in range(d):
          vals = plsc.load_gather(buf, [rows, cols[dd]])
          plsc.store_scatter(outv, [tdv[dd], tb, d8v[dd], b128], vals)
        return carry

      lax.fori_loop(0, _CB // nl, tj, 0)

    def unit(u, p, prefetch, first):
      wait(p)

      @pl.when(prefetch)
      def _():
        fire(u + 1, 1 - p)

      @pl.when(jnp.logical_not(first))
      def _():
        wait_wb(p)

      transpose(p)
      fire_wb(u, p)

    npairs = f // 2
    fire(0, 0)

    def body(j, carry):
      unit(2 * j, 0, jnp.bool_(True), j == 0)
      unit(2 * j + 1, 1, j < npairs - 1, j == 0)
      return carry

    lax.fori_loop(0, npairs, body, 0)
    wait_wb(0)
    wait_wb(1)

  return gather


def kernel(ids, table):
  b, f = ids.shape
  v, d = table.shape
  ids3 = ids.T.reshape(f, b // _IW, _IW)
  nfull = (v // _TC) * _TC
  tail_rm = table[nfull:, :].reshape((v - nfull) * d // 128, 128)
  packed = _make_pack(v, d)(table.T, tail_rm)
  table_rm = packed.reshape(v, d)
  out5 = _make_gather(b, f, v, d)(ids3, table_rm)
  return out5.transpose(2, 4, 0, 1, 3).reshape(b, f, d)


# speedup vs baseline: 11.9562x; 1.0020x over previous
import functools

import jax
import jax.numpy as jnp
from jax import lax
from jax.experimental import pallas as pl
from jax.experimental.pallas import tpu as pltpu
from jax.experimental.pallas import tpu_sc as plsc

_IW = 128
_RPU = 4
_CB = _RPU * _IW
_TC = 896


def _worker_id(nc):
  return lax.axis_index("s") * nc + lax.axis_index("c")


@functools.cache
def _make_pack(v: int, d: int):
  info = plsc.get_sparse_core_info()
  nc, ns, nl = info.num_cores, info.num_subcores, info.num_lanes
  nw = nc * ns
  nch = v // _TC
  tail = v - nch * _TC
  pairs = (nch + 2 * nw - 1) // (2 * nw)
  rpc = _TC * d // 128

  mesh = plsc.VectorSubcoreMesh(core_axis_name="c", subcore_axis_name="s")

  @functools.partial(
      pl.kernel,
      mesh=mesh,
      out_type=jax.ShapeDtypeStruct((v * d // 128, 128), jnp.float32),
      compiler_params=pltpu.CompilerParams(needs_layout_passes=False),
      scratch_types=[
          pltpu.VMEM((d, _TC), jnp.float32),
          pltpu.VMEM((d, _TC), jnp.float32),
          pltpu.VMEM((rpc, 128), jnp.float32),
          pltpu.VMEM((rpc, 128), jnp.float32),
          pltpu.SemaphoreType.DMA,
          pltpu.SemaphoreType.DMA,
          pltpu.SemaphoreType.DMA,
          pltpu.SemaphoreType.DMA,
      ],
  )
  def pack(tt_hbm, tail_hbm, out_hbm, buf0, buf1, ov0, ov1,
           ss0, ss1, sw0, sw1):
    wid = _worker_id(nc)
    lane = lax.iota(jnp.int32, nl)
    cols = [(lane + dd) & (d - 1) for dd in range(d)]
    bufs, ovs = (buf0, buf1), (ov0, ov1)
    ssems, wsems = (ss0, ss1), (sw0, sw1)

    def fire_stage(c, p):
      pltpu.async_copy(tt_hbm.at[:, pl.ds(c * _TC, _TC)], bufs[p], ssems[p])

    def wait_stage(p):
      pltpu.make_async_copy(
          tt_hbm.at[:, pl.ds(0, _TC)], bufs[p], ssems[p]).wait()

    def fire_wb(c, p):
      pltpu.async_copy(ovs[p], out_hbm.at[pl.ds(c * rpc, rpc), :], wsems[p])

    def wait_wb(p):
      pltpu.make_async_copy(
          ovs[p], out_hbm.at[pl.ds(0, rpc), :], wsems[p]).wait()

    def transpose(p):
      buf, outv = bufs[p], ovs[p]

      def tj(j, cc):
        rows = j * nl + lane
        rdiv = rows >> 2
        rmod = (rows & 3) << 5
        for dd in range(d):
          vals = plsc.load_gather(buf, [cols[dd], rows])
          plsc.store_scatter(outv, [rdiv, rmod + cols[dd]], vals)
        return cc

      lax.fori_loop(0, _TC // nl, tj, 0)

    def half(c_this, c_next2, p, first):
      @pl.when(c_this < nch)
      def _():
        wait_stage(p)

        @pl.when(jnp.logical_not(first))
        def _():
          wait_wb(p)

        transpose(p)
        fire_wb(c_this, p)

        @pl.when(c_next2 < nch)
        def _():
          fire_stage(c_next2, p)

    fire_stage(wid, 0)

    @pl.when(wid + nw < nch)
    def _():
      fire_stage(wid + nw, 1)

    def body(i, carry):
      ca = wid + (2 * i) * nw
      cb = wid + (2 * i + 1) * nw
      half(ca, wid + (2 * i + 2) * nw, 0, i == 0)
      half(cb, wid + (2 * i + 3) * nw, 1, i == 0)
      return carry

    lax.fori_loop(0, pairs, body, 0)
    wait_wb(0)
    wait_wb(1)

    if tail:
      trows = tail * d // 128

      @pl.when(wid == 0)
      def _():
        pltpu.sync_copy(tail_hbm, ov0.at[pl.ds(0, trows), :])
        pltpu.sync_copy(ov0.at[pl.ds(0, trows), :],
                        out_hbm.at[pl.ds(nch * rpc, trows), :])

  return pack


@functools.cache
def _make_gather(b: int, f: int, v: int, d: int):
  info = plsc.get_sparse_core_info()
  nc, ns, nl = info.num_cores, info.num_subcores, info.num_lanes
  nw = nc * ns
  assert b == nw * _CB and d % nl == 0 and nl == 16

  mesh = plsc.VectorSubcoreMesh(core_axis_name="c", subcore_axis_name="s")

  @functools.partial(
      pl.kernel,
      mesh=mesh,
      out_type=jax.ShapeDtypeStruct((f, d // 8, b // 128, 8, 128),
                                    jnp.float32),
      compiler_params=pltpu.CompilerParams(
          use_tc_tiling_on_sc=False, needs_layout_passes=False),
      scratch_types=[
          pltpu.VMEM((f, _RPU, _IW), jnp.int32),
          pltpu.VMEM((_CB, d), jnp.float32),
          pltpu.VMEM((_CB, d), jnp.float32),
          pltpu.VMEM((d // 8, _CB // 128, 8, 128), jnp.float32),
          pltpu.VMEM((d // 8, _CB // 128, 8, 128), jnp.float32),
          pltpu.SemaphoreType.DMA,
          pltpu.SemaphoreType.DMA,
          pltpu.SemaphoreType.DMA,
          pltpu.SemaphoreType.DMA,
      ],
  )
  def gather(idx_hbm, table_hbm, out_hbm, idx_all, buf0, buf1, ov0, ov1,
             sg0, sg1, sw0, sw1):
    wid = _worker_id(nc)
    b0 = wid * _CB
    pltpu.sync_copy(idx_hbm.at[:, pl.ds(wid * _RPU, _RPU), :], idx_all)

    bufs, ovs = (buf0, buf1), (ov0, ov1)
    gsems, wsems = (sg0, sg1), (sw0, sw1)
    lane = lax.iota(jnp.int32, nl)
    cols = [(lane + dd) & (d - 1) for dd in range(d)]

    def fire(u, p):
      for r in range(_RPU):
        pltpu.async_copy(
            table_hbm.at[idx_all.at[u, r]],
            bufs[p].at[pl.ds(r * _IW, _IW), :],
            gsems[p],
        )

    def wait(p):
      for r in range(_RPU):
        pltpu.make_async_copy(
            table_hbm.at[idx_all.at[0, r]],
            bufs[p].at[pl.ds(r * _IW, _IW), :],
            gsems[p],
        ).wait()

    tb0 = wid * (_CB // 128)
    tdv = [c >> 3 for c in cols]
    d8v = [c & 7 for c in cols]

    def fire_wb(u, p):
      pltpu.async_copy(
          ovs[p], out_hbm.at[u, :, pl.ds(tb0, _CB // 128), :, :], wsems[p])

    def wait_wb(p):
      pltpu.make_async_copy(
          ovs[p], out_hbm.at[0, :, pl.ds(0, _CB // 128), :, :],
          wsems[p]).wait()

    def transpose(p):
      buf, outv = bufs[p], ovs[p]

      def tj(j, carry):
        rows = j * nl + lane
        tb = rows >> 7
        b128 = rows & 127
        for dd in range(d):
          vals = plsc.load_gather(buf, [rows, cols[dd]])
          plsc.store_scatter(outv, [tdv[dd], tb, d8v[dd], b128], vals)
        return carry

      lax.fori_loop(0, _CB // nl, tj, 0)

    def unit(u, p, prefetch, first):
      wait(p)

      @pl.when(prefetch)
      def _():
        fire(u + 1, 1 - p)

      @pl.when(jnp.logical_not(first))
      def _():
        wait_wb(p)

      transpose(p)
      fire_wb(u, p)

    npairs = f // 2
    fire(0, 0)

    def body(j, carry):
      unit(2 * j, 0, jnp.bool_(True), j == 0)
      unit(2 * j + 1, 1, j < npairs - 1, j == 0)
      return carry

    lax.fori_loop(0, npairs, body, 0)
    wait_wb(0)
    wait_wb(1)

  return gather


def kernel(ids, table):
  b, f = ids.shape
  v, d = table.shape
  ids3 = ids.T.reshape(f, b // _IW, _IW)
  nfull = (v // _TC) * _TC
  tail_rm = table[nfull:, :].reshape((v - nfull) * d // 128, 128)
  packed = _make_pack(v, d)(table.T, tail_rm)
  table_rm = packed.reshape(v, d)
  out5 = _make_gather(b, f, v, d)(ids3, table_rm)
  return out5.transpose(2, 4, 0, 1, 3).reshape(b, f, d)


# speedup vs baseline: 28.4306x; 2.3779x over previous
import functools

import jax
import jax.numpy as jnp
from jax import lax
from jax.experimental import pallas as pl
from jax.experimental.pallas import tpu as pltpu
from jax.experimental.pallas import tpu_sc as plsc

_IW = 128
_RPU = 4
_CB = _RPU * _IW
_TC = 896


def _worker_id(nc):
  return lax.axis_index("s") * nc + lax.axis_index("c")


@functools.cache
def _make_pack(v: int, d: int):
  info = plsc.get_sparse_core_info()
  nc, ns, nl = info.num_cores, info.num_subcores, info.num_lanes
  nw = nc * ns
  nch = v // _TC
  tail = v - nch * _TC
  pairs = (nch + 2 * nw - 1) // (2 * nw)
  rpc = _TC * d // 128

  mesh = plsc.VectorSubcoreMesh(core_axis_name="c", subcore_axis_name="s")

  @functools.partial(
      pl.kernel,
      mesh=mesh,
      out_type=jax.ShapeDtypeStruct((v * d // 128, 128), jnp.float32),
      compiler_params=pltpu.CompilerParams(needs_layout_passes=False),
      scratch_types=[
          pltpu.VMEM((d, _TC), jnp.float32),
          pltpu.VMEM((d, _TC), jnp.float32),
          pltpu.VMEM((rpc, 128), jnp.float32),
          pltpu.VMEM((rpc, 128), jnp.float32),
          pltpu.SemaphoreType.DMA,
          pltpu.SemaphoreType.DMA,
          pltpu.SemaphoreType.DMA,
          pltpu.SemaphoreType.DMA,
      ],
  )
  def pack(tt_hbm, tail_hbm, out_hbm, buf0, buf1, ov0, ov1,
           ss0, ss1, sw0, sw1):
    wid = _worker_id(nc)
    lane = lax.iota(jnp.int32, nl)
    cols = [(lane + dd) & (d - 1) for dd in range(d)]
    bufs, ovs = (buf0, buf1), (ov0, ov1)
    ssems, wsems = (ss0, ss1), (sw0, sw1)

    def fire_stage(c, p):
      pltpu.async_copy(tt_hbm.at[:, pl.ds(c * _TC, _TC)], bufs[p], ssems[p])

    def wait_stage(p):
      pltpu.make_async_copy(
          tt_hbm.at[:, pl.ds(0, _TC)], bufs[p], ssems[p]).wait()

    def fire_wb(c, p):
      pltpu.async_copy(ovs[p], out_hbm.at[pl.ds(c * rpc, rpc), :], wsems[p])

    def wait_wb(p):
      pltpu.make_async_copy(
          ovs[p], out_hbm.at[pl.ds(0, rpc), :], wsems[p]).wait()

    def transpose(p):
      buf, outv = bufs[p], ovs[p]

      @functools.partial(plsc.parallel_loop, 0, _TC // nl, unroll=2)
      def _(j):
        rows = j * nl + lane
        rdiv = rows >> 2
        rmod = (rows & 3) << 5
        for dd in range(d):
          vals = plsc.load_gather(buf, [cols[dd], rows])
          plsc.store_scatter(outv, [rdiv, rmod + cols[dd]], vals)

    def half(c_this, c_next2, p, first):
      @pl.when(c_this < nch)
      def _():
        wait_stage(p)

        @pl.when(jnp.logical_not(first))
        def _():
          wait_wb(p)

        transpose(p)
        fire_wb(c_this, p)

        @pl.when(c_next2 < nch)
        def _():
          fire_stage(c_next2, p)

    fire_stage(wid, 0)

    @pl.when(wid + nw < nch)
    def _():
      fire_stage(wid + nw, 1)

    def body(i, carry):
      ca = wid + (2 * i) * nw
      cb = wid + (2 * i + 1) * nw
      half(ca, wid + (2 * i + 2) * nw, 0, i == 0)
      half(cb, wid + (2 * i + 3) * nw, 1, i == 0)
      return carry

    lax.fori_loop(0, pairs, body, 0)
    wait_wb(0)
    wait_wb(1)

    if tail:
      trows = tail * d // 128

      @pl.when(wid == 0)
      def _():
        pltpu.sync_copy(tail_hbm, ov0.at[pl.ds(0, trows), :])
        pltpu.sync_copy(ov0.at[pl.ds(0, trows), :],
                        out_hbm.at[pl.ds(nch * rpc, trows), :])

  return pack


@functools.cache
def _make_gather(b: int, f: int, v: int, d: int):
  info = plsc.get_sparse_core_info()
  nc, ns, nl = info.num_cores, info.num_subcores, info.num_lanes
  nw = nc * ns
  assert b == nw * _CB and d % nl == 0 and nl == 16

  mesh = plsc.VectorSubcoreMesh(core_axis_name="c", subcore_axis_name="s")

  @functools.partial(
      pl.kernel,
      mesh=mesh,
      out_type=jax.ShapeDtypeStruct((f, d // 8, b // 128, 8, 128),
                                    jnp.float32),
      compiler_params=pltpu.CompilerParams(
          use_tc_tiling_on_sc=False, needs_layout_passes=False),
      scratch_types=[
          pltpu.VMEM((f, _RPU, _IW), jnp.int32),
          pltpu.VMEM((_CB, d), jnp.float32),
          pltpu.VMEM((_CB, d), jnp.float32),
          pltpu.VMEM((d // 8, _CB // 128, 8, 128), jnp.float32),
          pltpu.VMEM((d // 8, _CB // 128, 8, 128), jnp.float32),
          pltpu.SemaphoreType.DMA,
          pltpu.SemaphoreType.DMA,
          pltpu.SemaphoreType.DMA,
          pltpu.SemaphoreType.DMA,
      ],
  )
  def gather(idx_hbm, table_hbm, out_hbm, idx_all, buf0, buf1, ov0, ov1,
             sg0, sg1, sw0, sw1):
    wid = _worker_id(nc)
    b0 = wid * _CB
    pltpu.sync_copy(idx_hbm.at[:, pl.ds(wid * _RPU, _RPU), :], idx_all)

    bufs, ovs = (buf0, buf1), (ov0, ov1)
    gsems, wsems = (sg0, sg1), (sw0, sw1)
    lane = lax.iota(jnp.int32, nl)
    cols = [(lane + dd) & (d - 1) for dd in range(d)]

    def fire(u, p):
      for r in range(_RPU):
        pltpu.async_copy(
            table_hbm.at[idx_all.at[u, r]],
            bufs[p].at[pl.ds(r * _IW, _IW), :],
            gsems[p],
        )

    def wait(p):
      for r in range(_RPU):
        pltpu.make_async_copy(
            table_hbm.at[idx_all.at[0, r]],
            bufs[p].at[pl.ds(r * _IW, _IW), :],
            gsems[p],
        ).wait()

    tb0 = wid * (_CB // 128)
    tdv = [c >> 3 for c in cols]
    d8v = [c & 7 for c in cols]

    def fire_wb(u, p):
      pltpu.async_copy(
          ovs[p], out_hbm.at[u, :, pl.ds(tb0, _CB // 128), :, :], wsems[p])

    def wait_wb(p):
      pltpu.make_async_copy(
          ovs[p], out_hbm.at[0, :, pl.ds(0, _CB // 128), :, :],
          wsems[p]).wait()

    def transpose(p):
      buf, outv = bufs[p], ovs[p]

      @functools.partial(plsc.parallel_loop, 0, _CB // nl, unroll=2)
      def _(j):
        rows = j * nl + lane
        tb = rows >> 7
        b128 = rows & 127
        for dd in range(d):
          vals = plsc.load_gather(buf, [rows, cols[dd]])
          plsc.store_scatter(outv, [tdv[dd], tb, d8v[dd], b128], vals)

    def unit(u, p, prefetch, first):
      wait(p)

      @pl.when(prefetch)
      def _():
        fire(u + 1, 1 - p)

      @pl.when(jnp.logical_not(first))
      def _():
        wait_wb(p)

      transpose(p)
      fire_wb(u, p)

    npairs = f // 2
    fire(0, 0)

    def body(j, carry):
      unit(2 * j, 0, jnp.bool_(True), j == 0)
      unit(2 * j + 1, 1, j < npairs - 1, j == 0)
      return carry

    lax.fori_loop(0, npairs, body, 0)
    wait_wb(0)
    wait_wb(1)

  return gather


def kernel(ids, table):
  b, f = ids.shape
  v, d = table.shape
  ids3 = ids.T.reshape(f, b // _IW, _IW)
  nfull = (v // _TC) * _TC
  tail_rm = table[nfull:, :].reshape((v - nfull) * d // 128, 128)
  packed = _make_pack(v, d)(table.T, tail_rm)
  table_rm = packed.reshape(v, d)
  out5 = _make_gather(b, f, v, d)(ids3, table_rm)
  return out5.transpose(2, 4, 0, 1, 3).reshape(b, f, d)
